# baseline (speedup 1.0000x reference)
import functools

import jax
import jax.numpy as jnp
from jax import lax
from jax.experimental import pallas as pl
from jax.experimental.pallas import tpu as pltpu
from jax.experimental.pallas import tpu_sc as plsc

_B = 16384
_D = 32
_N = 1000000
_W = 128
_NSLAB = (_N + _W - 1) // _W
_GROWS = ((_NSLAB * _W) >> 2)
_NC, _NS = 2, 16
_NW = _NC * _NS
_BPW = _B // _NW
_CH = 128
_NCH = _BPW // _CH

_mesh = functools.partial(plsc.VectorSubcoreMesh,
                          core_axis_name="c", subcore_axis_name="s",
                          num_cores=_NC, num_subcores=_NS)


@functools.lru_cache(maxsize=None)
def _relayout_kernel():
    @functools.partial(
        pl.kernel,
        mesh=_mesh(),
        out_type=(
            jax.ShapeDtypeStruct((_GROWS, _W), jnp.float32),
            jax.ShapeDtypeStruct((_GROWS, _W), jnp.float32),
        ),
        scratch_types=[
            pltpu.VMEM((_D, _W), jnp.float32),
            pltpu.VMEM((_D, _W), jnp.float32),
            pltpu.VMEM((_D, _W), jnp.float32),
            pltpu.VMEM((_D, _W), jnp.float32),
        ],
        compiler_params=pltpu.CompilerParams(use_tc_tiling_on_sc=True,
                                             needs_layout_passes=False),
    )
    def _relayout(utt_hbm, itt_hbm, gu_hbm, gi_hbm,
                  uin_v, iin_v, uout_v, iout_v):
        wid = lax.axis_index("s") * _NC + lax.axis_index("c")
        count = (_NSLAB + _NW - 1 - wid) // _NW
        f0 = lax.iota(jnp.int32, 16)
        f1 = f0 + 16

        def slab(i, carry):
            j = wid + i * _NW
            pltpu.sync_copy(utt_hbm.at[:, pl.ds(j * _W, _W)], uin_v)
            pltpu.sync_copy(itt_hbm.at[:, pl.ds(j * _W, _W)], iin_v)
            for g in range(_D):
                for s in range(4):
                    col = jnp.full((16,), 4 * g + s, jnp.int32)
                    uout_v[g, pl.ds(s * _D, 16)] = plsc.load_gather(
                        uin_v, [f0, col])
                    uout_v[g, pl.ds(s * _D + 16, 16)] = plsc.load_gather(
                        uin_v, [f1, col])
                    iout_v[g, pl.ds(s * _D, 16)] = plsc.load_gather(
                        iin_v, [f0, col])
                    iout_v[g, pl.ds(s * _D + 16, 16)] = plsc.load_gather(
                        iin_v, [f1, col])
            pltpu.sync_copy(uout_v, gu_hbm.at[pl.ds(j * _D, _D)])
            pltpu.sync_copy(iout_v, gi_hbm.at[pl.ds(j * _D, _D)])
            return carry

        lax.fori_loop(0, count, slab, 0)

    return _relayout


@functools.lru_cache(maxsize=None)
def _gather_kernel():
    @functools.partial(
        pl.kernel,
        mesh=_mesh(),
        out_type=(
            jax.ShapeDtypeStruct((_B, _W), jnp.float32),
            jax.ShapeDtypeStruct((_B, _W), jnp.float32),
        ),
        scratch_types=[
            pltpu.VMEM((_NCH, _CH), jnp.int32),
            pltpu.VMEM((_NCH, _CH), jnp.int32),
            pltpu.VMEM((_CH, _W), jnp.float32),
            pltpu.VMEM((_CH, _W), jnp.float32),
            pltpu.SemaphoreType.DMA,
            pltpu.SemaphoreType.DMA,
        ],
        compiler_params=pltpu.CompilerParams(use_tc_tiling_on_sc=True),
    )
    def _gather(ugid_hbm, igid_hbm, gu_hbm, gi_hbm,
                uout_hbm, iout_hbm,
                uidx_v, iidx_v, urows_v, irows_v, usem, isem):
        wid = lax.axis_index("s") * _NC + lax.axis_index("c")
        base = wid * _BPW
        for j in range(_NCH):
            pltpu.sync_copy(ugid_hbm.at[pl.ds(base + j * _CH, _CH)],
                            uidx_v.at[j])
            pltpu.sync_copy(igid_hbm.at[pl.ds(base + j * _CH, _CH)],
                            iidx_v.at[j])
        for j in range(_NCH):
            gu = pltpu.async_copy(gu_hbm.at[uidx_v.at[j]], urows_v, usem)
            gi = pltpu.async_copy(gi_hbm.at[iidx_v.at[j]], irows_v, isem)
            gu.wait()
            pltpu.sync_copy(urows_v, uout_hbm.at[pl.ds(base + j * _CH, _CH)])
            gi.wait()
            pltpu.sync_copy(irows_v, iout_hbm.at[pl.ds(base + j * _CH, _CH)])

    return _gather


_BM = 2048


def _mlp_body(u_ref, v_ref, usel_ref, vsel_ref, w1u_ref, w1v_ref, b1_ref,
              w2_ref, b2_ref, w3_ref, b3_ref, o_ref):
    sub = lax.broadcasted_iota(jnp.int32, (_BM, _W), 1) // _D
    xu = jnp.where(sub == usel_ref[...], u_ref[...], 0.0)
    xv = jnp.where(sub == vsel_ref[...], v_ref[...], 0.0)
    x1 = (jnp.dot(xu, w1u_ref[...], preferred_element_type=jnp.float32)
          + jnp.dot(xv, w1v_ref[...], preferred_element_type=jnp.float32)
          + b1_ref[...])
    h1 = jnp.maximum(x1, 0.0)
    h2 = jnp.maximum(
        jnp.dot(h1, w2_ref[...], preferred_element_type=jnp.float32)
        + b2_ref[...], 0.0)
    o_ref[...] = (jnp.dot(h2, w3_ref[...], preferred_element_type=jnp.float32)
                  + b3_ref[...])


def _mlp(u128, i128, usel, isel, W1u4, W1i4, b1, W2, b2, W3, b3):
    return pl.pallas_call(
        _mlp_body,
        grid=(_B // _BM,),
        in_specs=[
            pl.BlockSpec((_BM, _W), lambda m: (m, 0)),
            pl.BlockSpec((_BM, _W), lambda m: (m, 0)),
            pl.BlockSpec((_BM, 1), lambda m: (m, 0)),
            pl.BlockSpec((_BM, 1), lambda m: (m, 0)),
            pl.BlockSpec((_W, 64), lambda m: (0, 0)),
            pl.BlockSpec((_W, 64), lambda m: (0, 0)),
            pl.BlockSpec((1, 64), lambda m: (0, 0)),
            pl.BlockSpec((64, 32), lambda m: (0, 0)),
            pl.BlockSpec((1, 32), lambda m: (0, 0)),
            pl.BlockSpec((32, 1), lambda m: (0, 0)),
            pl.BlockSpec((1, 1), lambda m: (0, 0)),
        ],
        out_specs=pl.BlockSpec((_BM, 1), lambda m: (m, 0)),
        out_shape=jax.ShapeDtypeStruct((_B, 1), jnp.float32),
    )(u128, i128, usel, isel, W1u4, W1i4, b1.reshape(1, 64),
      W2, b2.reshape(1, 32), W3, b3.reshape(1, 1))


def kernel(inputs, user_table, item_table, W1, b1, W2, b2, W3, b3):
    idx = inputs.astype(jnp.int32)
    ugid = idx[:, 0] >> 2
    igid = idx[:, 1] >> 2
    usel = (idx[:, 0] & 3).reshape(_B, 1)
    isel = (idx[:, 1] & 3).reshape(_B, 1)
    gu, gi = _relayout_kernel()(user_table.T, item_table.T)
    u128, i128 = _gather_kernel()(ugid, igid, gu, gi)
    W1u4 = jnp.tile(W1[:_D, :], (4, 1))
    W1i4 = jnp.tile(W1[_D:, :], (4, 1))
    return _mlp(u128, i128, usel, isel, W1u4, W1i4, b1, W2, b2, W3, b3)
